# Initial kernel scaffold; baseline (speedup 1.0000x reference)
import jax
import jax.numpy as jnp
from jax.experimental import pallas as pl


def kernel(node_states, edge_index, W0, b0, W1, b1, W2, b2):
    raise NotImplementedError("write your pallas kernel here")


# speedup vs baseline: 7.3897x; 7.3897x over previous
import functools

import jax
import jax.numpy as jnp
from jax import lax
from jax.experimental import pallas as pl
from jax.experimental.pallas import tpu as pltpu
from jax.experimental.pallas import tpu_sc as plsc

_N = 10000
_E = 320000
_D = 128
_NC, _NS = 2, 16
_NW = _NC * _NS
_EPW = _E // _NW
_CH = 125
_NCHUNK = _EPW // _CH
_BLK = 8
_WRS = 624
_TAIL = _N - _NS * _WRS
_ZR = 48
_DG = 16
_BR = 1000

_MESH = plsc.VectorSubcoreMesh(core_axis_name="c", subcore_axis_name="s")


def _fill(ref, rows, width, value):
    def row(i, carry):
        def col(k, c2):
            ref[i, pl.ds(k * 16, 16)] = jnp.full((16,), value, jnp.float32)
            return c2
        return lax.fori_loop(0, width // 16, col, carry)
    lax.fori_loop(0, rows, row, 0)


def _zero_slabs(s, sh_ref, z_ref):
    r0 = s * _WRS
    for t in range(_WRS // _ZR):
        pltpu.sync_copy(z_ref, sh_ref.at[pl.ds(r0 + t * _ZR, _ZR)])

    @pl.when(s == _NS - 1)
    def _tail():
        pltpu.sync_copy(z_ref.at[pl.ds(0, _TAIL)],
                        sh_ref.at[pl.ds(_NS * _WRS, _TAIL)])


def _write_slabs(s, c, sh_ref, out_ref):
    r0 = s * _WRS
    pltpu.sync_copy(sh_ref.at[pl.ds(r0, _WRS)],
                    out_ref.at[c].at[pl.ds(r0, _WRS)])

    @pl.when(s == _NS - 1)
    def _tail():
        pltpu.sync_copy(sh_ref.at[pl.ds(_NS * _WRS, _TAIL)],
                        out_ref.at[c].at[pl.ds(_NS * _WRS, _TAIL)])


def _sc_agg_body(x_hbm, src_hbm, dst_hbm, agg_out,
                 srcb_v, dstb_v, rows_v, zb_v, sem, agg_sh):
    c = lax.axis_index("c")
    s = lax.axis_index("s")
    wid = c * _NS + s
    _fill(zb_v, _ZR, _D, 0.0)
    _zero_slabs(s, agg_sh, zb_v)
    plsc.subcore_barrier()

    def step(j, carry):
        p = j % _BLK

        @pl.when(p == 0)
        def _load_idx():
            off = pl.multiple_of(wid * _NCHUNK + j, _BLK)
            pltpu.sync_copy(src_hbm.at[pl.ds(off, _BLK)], srcb_v)
            pltpu.sync_copy(dst_hbm.at[pl.ds(off, _BLK)], dstb_v)

        pltpu.async_copy(x_hbm.at[srcb_v.at[p]], rows_v, sem).wait()
        pltpu.sync_copy(rows_v, agg_sh.at[dstb_v.at[p]], add=True)
        return carry

    lax.fori_loop(0, _NCHUNK, step, 0)
    plsc.subcore_barrier()
    _write_slabs(s, c, agg_sh, agg_out)


def _sc_deg_body(dst_hbm, deg_out, dstb_v, zd_v, ones_v, deg_sh):
    c = lax.axis_index("c")
    s = lax.axis_index("s")
    wid = c * _NS + s
    _fill(zd_v, _ZR, _D, 0.0)
    _fill(ones_v, _CH, _D, 1.0)
    _zero_slabs(s, deg_sh, zd_v)
    plsc.subcore_barrier()

    def step(j, carry):
        p = j % _BLK

        @pl.when(p == 0)
        def _load_idx():
            off = pl.multiple_of(wid * _NCHUNK + j, _BLK)
            pltpu.sync_copy(dst_hbm.at[pl.ds(off, _BLK)], dstb_v)

        pltpu.sync_copy(ones_v, deg_sh.at[dstb_v.at[p]], add=True)
        return carry

    lax.fori_loop(0, _NCHUNK, step, 0)
    plsc.subcore_barrier()
    _write_slabs(s, c, deg_sh, deg_out)


_sc_agg = functools.partial(
    pl.kernel,
    out_type=[jax.ShapeDtypeStruct((_NC, _N, _D), jnp.float32)],
    mesh=_MESH,
    scratch_types=[
        pltpu.VMEM((_BLK, _CH), jnp.int32),
        pltpu.VMEM((_BLK, _CH), jnp.int32),
        pltpu.VMEM((_CH, _D), jnp.float32),
        pltpu.VMEM((_ZR, _D), jnp.float32),
        pltpu.SemaphoreType.DMA,
        pltpu.VMEM_SHARED((_N, _D), jnp.float32),
    ],
)(_sc_agg_body)


_sc_deg = functools.partial(
    pl.kernel,
    out_type=[jax.ShapeDtypeStruct((_NC, _N, _D), jnp.float32)],
    mesh=_MESH,
    scratch_types=[
        pltpu.VMEM((_BLK, _CH), jnp.int32),
        pltpu.VMEM((_ZR, _D), jnp.float32),
        pltpu.VMEM((_CH, _D), jnp.float32),
        pltpu.VMEM_SHARED((_N, _D), jnp.float32),
    ],
)(_sc_deg_body)


def _linear_body(x_ref, agg0_ref, agg1_ref, deg0_ref, deg1_ref,
                 wt_ref, wb_ref, b_ref, o_ref, *, relu):
    deg = deg0_ref[:, 0:1] + deg1_ref[:, 0:1]
    safe = jnp.where(deg > 0, deg, 1.0)
    dinv = jnp.where(deg > 0, 1.0 / safe, 0.0)
    neigh = (agg0_ref[...] + agg1_ref[...]) * dinv
    h = jnp.dot(x_ref[...], wt_ref[...], preferred_element_type=jnp.float32)
    h = h + jnp.dot(neigh, wb_ref[...], preferred_element_type=jnp.float32)
    h = h + b_ref[...]
    o_ref[...] = jnp.maximum(h, 0.0) if relu else h


def _linear(x, agg, deg, wt, wb, b2d, relu):
    row_spec = pl.BlockSpec((_BR, _D), lambda i: (i, 0))
    deg_spec = pl.BlockSpec((_BR, _D), lambda i: (i, 0))
    full_spec = pl.BlockSpec((_D, _D), lambda i: (0, 0))
    return pl.pallas_call(
        functools.partial(_linear_body, relu=relu),
        grid=(_N // _BR,),
        in_specs=[row_spec, row_spec, row_spec, deg_spec, deg_spec,
                  full_spec, full_spec,
                  pl.BlockSpec((1, _D), lambda i: (0, 0))],
        out_specs=row_spec,
        out_shape=jax.ShapeDtypeStruct((_N, _D), jnp.float32),
    )(x, agg[0], agg[1], deg[0], deg[1], wt, wb, b2d)


def kernel(node_states, edge_index, W0, b0, W1, b1, W2, b2):
    dst2d = edge_index[0].astype(jnp.int32).reshape(_NW * _NCHUNK, _CH)
    src2d = edge_index[1].astype(jnp.int32).reshape(_NW * _NCHUNK, _CH)

    (deg,) = _sc_deg(dst2d)
    (agg0,) = _sc_agg(node_states, src2d, dst2d)
    h1 = _linear(node_states, agg0, deg, W0[:_D], W0[_D:],
                 b0.reshape(1, _D), True)
    (agg1,) = _sc_agg(h1, src2d, dst2d)
    h2 = _linear(h1, agg1, deg, W1[:_D], W1[_D:], b1.reshape(1, _D), True)
    (agg2,) = _sc_agg(h2, src2d, dst2d)
    return _linear(h2, agg2, deg, W2[:_D], W2[_D:], b2.reshape(1, _D), False)


# speedup vs baseline: 9.1351x; 1.2362x over previous
import functools

import jax
import jax.numpy as jnp
from jax import lax
from jax.experimental import pallas as pl
from jax.experimental.pallas import tpu as pltpu
from jax.experimental.pallas import tpu_sc as plsc

_N = 10000
_E = 320000
_D = 128
_NC, _NS = 2, 16
_NW = _NC * _NS
_EPW = _E // _NW
_CH = 125
_NCHUNK = _EPW // _CH
_BLK = 8
_WRS = 624
_TAIL = _N - _NS * _WRS
_ZR = 48
_DG = 16
_BR = 1000

_MESH = plsc.VectorSubcoreMesh(core_axis_name="c", subcore_axis_name="s")


def _fill(ref, rows, width, value):
    def row(i, carry):
        def col(k, c2):
            ref[i, pl.ds(k * 16, 16)] = jnp.full((16,), value, jnp.float32)
            return c2
        return lax.fori_loop(0, width // 16, col, carry)
    lax.fori_loop(0, rows, row, 0)


def _zero_slabs(s, sh_ref, z_ref):
    r0 = s * _WRS
    for t in range(_WRS // _ZR):
        pltpu.sync_copy(z_ref, sh_ref.at[pl.ds(r0 + t * _ZR, _ZR)])

    @pl.when(s == _NS - 1)
    def _tail():
        pltpu.sync_copy(z_ref.at[pl.ds(0, _TAIL)],
                        sh_ref.at[pl.ds(_NS * _WRS, _TAIL)])


def _write_slabs(s, c, sh_ref, out_ref):
    r0 = s * _WRS
    pltpu.sync_copy(sh_ref.at[pl.ds(r0, _WRS)],
                    out_ref.at[c].at[pl.ds(r0, _WRS)])

    @pl.when(s == _NS - 1)
    def _tail():
        pltpu.sync_copy(sh_ref.at[pl.ds(_NS * _WRS, _TAIL)],
                        out_ref.at[c].at[pl.ds(_NS * _WRS, _TAIL)])


def _sc_agg_body(x_hbm, src_hbm, dst_hbm, agg_out,
                 srcb_v, dstb_v, rows_v, zb_v, sem, agg_sh):
    c = lax.axis_index("c")
    s = lax.axis_index("s")
    wid = c * _NS + s
    _fill(zb_v, _ZR, _D, 0.0)
    _zero_slabs(s, agg_sh, zb_v)
    plsc.subcore_barrier()

    pltpu.sync_copy(src_hbm.at[pl.ds(wid * _NCHUNK, _BLK)], srcb_v.at[0])
    pltpu.sync_copy(dst_hbm.at[pl.ds(wid * _NCHUNK, _BLK)], dstb_v.at[0])
    pltpu.async_copy(x_hbm.at[srcb_v.at[0].at[0]], rows_v.at[0], sem)

    def step(j, carry):
        jb = j % 2
        p = j % _BLK
        bb = (j // _BLK) % 2
        pltpu.make_async_copy(x_hbm.at[srcb_v.at[bb].at[p]],
                              rows_v.at[jb], sem).wait()

        @pl.when(((j + 1) % _BLK == 0) & (j + 1 < _NCHUNK))
        def _load_next_idx():
            off = pl.multiple_of(wid * _NCHUNK + j + 1, _BLK)
            nbb = ((j + 1) // _BLK) % 2
            pltpu.sync_copy(src_hbm.at[pl.ds(off, _BLK)], srcb_v.at[nbb])
            pltpu.sync_copy(dst_hbm.at[pl.ds(off, _BLK)], dstb_v.at[nbb])

        @pl.when(j + 1 < _NCHUNK)
        def _prefetch():
            p1 = (j + 1) % _BLK
            bb1 = ((j + 1) // _BLK) % 2
            pltpu.async_copy(x_hbm.at[srcb_v.at[bb1].at[p1]],
                             rows_v.at[(j + 1) % 2], sem)

        pltpu.sync_copy(rows_v.at[jb], agg_sh.at[dstb_v.at[bb].at[p]],
                        add=True)
        return carry

    lax.fori_loop(0, _NCHUNK, step, 0)
    plsc.subcore_barrier()
    _write_slabs(s, c, agg_sh, agg_out)


def _sc_deg_body(dst_hbm, deg_out, dstb_v, zd_v, ones_v, deg_sh):
    c = lax.axis_index("c")
    s = lax.axis_index("s")
    wid = c * _NS + s
    _fill(zd_v, _ZR, _D, 0.0)
    _fill(ones_v, _CH, _D, 1.0)
    _zero_slabs(s, deg_sh, zd_v)
    plsc.subcore_barrier()

    def step(j, carry):
        p = j % _BLK

        @pl.when(p == 0)
        def _load_idx():
            off = pl.multiple_of(wid * _NCHUNK + j, _BLK)
            pltpu.sync_copy(dst_hbm.at[pl.ds(off, _BLK)], dstb_v)

        pltpu.sync_copy(ones_v, deg_sh.at[dstb_v.at[p]], add=True)
        return carry

    lax.fori_loop(0, _NCHUNK, step, 0)
    plsc.subcore_barrier()
    _write_slabs(s, c, deg_sh, deg_out)


_sc_agg = functools.partial(
    pl.kernel,
    out_type=[jax.ShapeDtypeStruct((_NC, _N, _D), jnp.float32)],
    mesh=_MESH,
    scratch_types=[
        pltpu.VMEM((2, _BLK, _CH), jnp.int32),
        pltpu.VMEM((2, _BLK, _CH), jnp.int32),
        pltpu.VMEM((2, _CH, _D), jnp.float32),
        pltpu.VMEM((_ZR, _D), jnp.float32),
        pltpu.SemaphoreType.DMA,
        pltpu.VMEM_SHARED((_N, _D), jnp.float32),
    ],
)(_sc_agg_body)


_sc_deg = functools.partial(
    pl.kernel,
    out_type=[jax.ShapeDtypeStruct((_NC, _N, _D), jnp.float32)],
    mesh=_MESH,
    scratch_types=[
        pltpu.VMEM((_BLK, _CH), jnp.int32),
        pltpu.VMEM((_ZR, _D), jnp.float32),
        pltpu.VMEM((_CH, _D), jnp.float32),
        pltpu.VMEM_SHARED((_N, _D), jnp.float32),
    ],
)(_sc_deg_body)


def _linear_body(x_ref, agg0_ref, agg1_ref, deg0_ref, deg1_ref,
                 wt_ref, wb_ref, b_ref, o_ref, *, relu):
    deg = deg0_ref[:, 0:1] + deg1_ref[:, 0:1]
    safe = jnp.where(deg > 0, deg, 1.0)
    dinv = jnp.where(deg > 0, 1.0 / safe, 0.0)
    neigh = (agg0_ref[...] + agg1_ref[...]) * dinv
    h = jnp.dot(x_ref[...], wt_ref[...], preferred_element_type=jnp.float32)
    h = h + jnp.dot(neigh, wb_ref[...], preferred_element_type=jnp.float32)
    h = h + b_ref[...]
    o_ref[...] = jnp.maximum(h, 0.0) if relu else h


def _linear(x, agg, deg, wt, wb, b2d, relu):
    row_spec = pl.BlockSpec((_BR, _D), lambda i: (i, 0))
    deg_spec = pl.BlockSpec((_BR, _D), lambda i: (i, 0))
    full_spec = pl.BlockSpec((_D, _D), lambda i: (0, 0))
    return pl.pallas_call(
        functools.partial(_linear_body, relu=relu),
        grid=(_N // _BR,),
        in_specs=[row_spec, row_spec, row_spec, deg_spec, deg_spec,
                  full_spec, full_spec,
                  pl.BlockSpec((1, _D), lambda i: (0, 0))],
        out_specs=row_spec,
        out_shape=jax.ShapeDtypeStruct((_N, _D), jnp.float32),
    )(x, agg[0], agg[1], deg[0], deg[1], wt, wb, b2d)


def kernel(node_states, edge_index, W0, b0, W1, b1, W2, b2):
    dst2d = edge_index[0].astype(jnp.int32).reshape(_NW * _NCHUNK, _CH)
    src2d = edge_index[1].astype(jnp.int32).reshape(_NW * _NCHUNK, _CH)

    (deg,) = _sc_deg(dst2d)
    (agg0,) = _sc_agg(node_states, src2d, dst2d)
    h1 = _linear(node_states, agg0, deg, W0[:_D], W0[_D:],
                 b0.reshape(1, _D), True)
    (agg1,) = _sc_agg(h1, src2d, dst2d)
    h2 = _linear(h1, agg1, deg, W1[:_D], W1[_D:], b1.reshape(1, _D), True)
    (agg2,) = _sc_agg(h2, src2d, dst2d)
    return _linear(h2, agg2, deg, W2[:_D], W2[_D:], b2.reshape(1, _D), False)
